# Initial kernel scaffold; baseline (speedup 1.0000x reference)
import jax
import jax.numpy as jnp
from jax.experimental import pallas as pl


def kernel(node_states, edges, kernel, kernel_attention):
    raise NotImplementedError("write your pallas kernel here")


# speedup vs baseline: 1.2148x; 1.2148x over previous
import functools

import jax
import jax.numpy as jnp
from jax import lax
from jax.experimental import pallas as pl
from jax.experimental.pallas import tpu as pltpu
from jax.experimental.pallas import tpu_sc as plsc

N_NODES = 10000
N_EDGES = 320000
D = 128

NC = 2
NS = 16
NW = NC * NS
N_PAD = 10240
RPT = N_PAD // NW
CC = 4000
N_CHUNKS = N_EDGES // CC
SUB = 128


def _prep_body(ns_ref, w_ref, a1_ref, a2_ref, h_ref, s1_ref, s2_ref):
    h = jnp.dot(ns_ref[...], w_ref[...], preferred_element_type=jnp.float32)
    h_ref[...] = h
    s1_ref[...] = jnp.dot(h, a1_ref[...], preferred_element_type=jnp.float32)
    s2_ref[...] = jnp.dot(h, a2_ref[...], preferred_element_type=jnp.float32)


def _prep(node_states, w, a1, a2):
    return pl.pallas_call(
        _prep_body,
        out_shape=(
            jax.ShapeDtypeStruct((N_NODES, D), jnp.float32),
            jax.ShapeDtypeStruct((N_NODES,), jnp.float32),
            jax.ShapeDtypeStruct((N_NODES,), jnp.float32),
        ),
    )(node_states, w, a1, a2)


def _sc_body(h_hbm, s1_hbm, s2_hbm, dst_hbm, src_hbm,
             num_out, den_out,
             s1r_v, s2_v, dstc_v, srcc_v, cd_v, cs_v, w_v, rows_v,
             acc_v, den_v, gsem):
    cid = lax.axis_index("c")
    sid = lax.axis_index("s")
    wid = sid * NC + cid
    lo = wid * RPT
    hi = lo + RPT

    pltpu.sync_copy(s1_hbm.at[pl.ds(lo, RPT)], s1r_v)
    pltpu.sync_copy(s2_hbm, s2_v)

    zeros = jnp.zeros((16,), jnp.float32)

    def _zero(i, _):
        for j in range(D // 16):
            acc_v[i, pl.ds(j * 16, 16)] = zeros
        den_v[i, :] = zeros
        return 0
    lax.fori_loop(0, RPT, _zero, 0)

    def _chunk(c, _):
        e0 = c * CC
        pltpu.sync_copy(dst_hbm.at[pl.ds(e0, CC)], dstc_v)
        pltpu.sync_copy(src_hbm.at[pl.ds(e0, CC)], srcc_v)

        def _cmp(g, cnt):
            sl = pl.ds(g * 16, 16)
            d16 = dstc_v[sl]
            s16 = srcc_v[sl]
            msk = (d16 >= lo) & (d16 < hi)
            plsc.store_compressed(cd_v.at[pl.ds(cnt, 16)], d16 - lo, mask=msk)
            plsc.store_compressed(cs_v.at[pl.ds(cnt, 16)], s16, mask=msk)
            return cnt + plsc.all_reduce_population_count(msk)[0]
        m = lax.fori_loop(0, CC // 16, _cmp, jnp.int32(0))

        for p in range(SUB // 16):
            cd_v[pl.ds(m + p * 16, 16)] = zeros.astype(jnp.int32)
            cs_v[pl.ds(m + p * 16, 16)] = zeros.astype(jnp.int32)

        def _sub(q, _):
            i0 = q * SUB
            gather = pltpu.async_copy(
                h_hbm.at[cs_v.at[pl.ds(i0, SUB)]], rows_v, gsem)
            for g in range(SUB // 16):
                sl16 = pl.ds(i0 + g * 16, 16)
                dl16 = cd_v[sl16]
                s16 = cs_v[sl16]
                x = (plsc.load_gather(s1r_v, [dl16])
                     + plsc.load_gather(s2_v, [s16]))
                x = jnp.where(x >= 0.0, x, 0.01 * x)
                x = jnp.minimum(jnp.maximum(x, -2.0), 2.0)
                lane = i0 + g * 16 + lax.iota(jnp.int32, 16)
                w_v[pl.ds(g * 16, 16)] = jnp.where(lane < m, jnp.exp(x), 0.0)
            gather.wait()

            def _grp(g2, _):
                dl16 = cd_v[pl.ds(i0 + g2 * 16, 16)]
                w16 = w_v[pl.ds(g2 * 16, 16)]
                for i in range(16):
                    dloc = dl16[i]
                    wv = jnp.full((16,), w16[i], jnp.float32)
                    e = g2 * 16 + i
                    den_v[dloc, :] = den_v[dloc, :] + wv
                    for j in range(D // 16):
                        sl = pl.ds(j * 16, 16)
                        acc_v[dloc, sl] = acc_v[dloc, sl] + wv * rows_v[e, sl]
                return 0
            lax.fori_loop(0, SUB // 16, _grp, 0)
            return 0

        lax.fori_loop(0, (m + SUB - 1) // SUB, _sub, 0)
        return 0

    lax.fori_loop(0, N_CHUNKS, _chunk, 0)

    pltpu.sync_copy(acc_v, num_out.at[pl.ds(lo, RPT)])
    pltpu.sync_copy(den_v, den_out.at[pl.ds(lo, RPT)])


def _sc_accum(h, s1p, s2p, dst, src):
    mesh = plsc.VectorSubcoreMesh(core_axis_name="c", subcore_axis_name="s",
                                  num_cores=NC)
    fn = functools.partial(
        pl.kernel,
        mesh=mesh,
        out_type=(
            jax.ShapeDtypeStruct((N_PAD, D), jnp.float32),
            jax.ShapeDtypeStruct((N_PAD, 16), jnp.float32),
        ),
        scratch_types=[
            pltpu.VMEM((RPT,), jnp.float32),
            pltpu.VMEM((N_PAD,), jnp.float32),
            pltpu.VMEM((CC,), jnp.int32),
            pltpu.VMEM((CC,), jnp.int32),
            pltpu.VMEM((CC + 2 * SUB,), jnp.int32),
            pltpu.VMEM((CC + 2 * SUB,), jnp.int32),
            pltpu.VMEM((SUB,), jnp.float32),
            pltpu.VMEM((SUB, D), jnp.float32),
            pltpu.VMEM((RPT, D), jnp.float32),
            pltpu.VMEM((RPT, 16), jnp.float32),
            pltpu.SemaphoreType.DMA,
        ],
        compiler_params=pltpu.CompilerParams(needs_layout_passes=False),
    )(_sc_body)
    return fn(h, s1p, s2p, dst, src)


def _fin_body(num_ref, den_ref, out_ref):
    den = den_ref[:, 0:1]
    den = jnp.where(den > 0.0, den, 1.0)
    out_ref[...] = num_ref[...] / den


def _finish(num, den):
    blk = 1000
    grid = N_NODES // blk
    return pl.pallas_call(
        _fin_body,
        grid=(grid,),
        in_specs=[
            pl.BlockSpec((blk, D), lambda i: (i, 0)),
            pl.BlockSpec((blk, 16), lambda i: (i, 0)),
        ],
        out_specs=pl.BlockSpec((blk, D), lambda i: (i, 0)),
        out_shape=jax.ShapeDtypeStruct((N_NODES, D), jnp.float32),
    )(num, den)


def kernel(node_states, edges, kernel, kernel_attention):
    a = kernel_attention[:, 0]
    a1 = a[:D]
    a2 = a[D:]
    h, s1, s2 = _prep(node_states, kernel, a1, a2)
    s1p = jnp.pad(s1, (0, N_PAD - N_NODES))
    s2p = jnp.pad(s2, (0, N_PAD - N_NODES))
    dst = edges[:, 0].astype(jnp.int32)
    src = edges[:, 1].astype(jnp.int32)
    num, den = _sc_accum(h, s1p, s2p, dst, src)
    return _finish(num, den)


# speedup vs baseline: 2.0230x; 1.6653x over previous
import functools

import jax
import jax.numpy as jnp
from jax import lax
from jax.experimental import pallas as pl
from jax.experimental.pallas import tpu as pltpu
from jax.experimental.pallas import tpu_sc as plsc

N_NODES = 10000
N_EDGES = 320000
D = 128

NC = 2
NS = 16
NW = NC * NS
N_PAD = 10240
RPT = N_PAD // NW
CC = 3200
N_CHUNKS = N_EDGES // CC
SUB = 128
CCP = CC + SUB + 16


def _prep_body(ns_ref, w_ref, a1_ref, a2_ref, h_ref, s1_ref, s2_ref):
    h = jnp.dot(ns_ref[...], w_ref[...], preferred_element_type=jnp.float32)
    h_ref[...] = h
    s1_ref[...] = jnp.dot(h, a1_ref[...], preferred_element_type=jnp.float32)
    s2_ref[...] = jnp.dot(h, a2_ref[...], preferred_element_type=jnp.float32)


def _prep(node_states, w, a1, a2):
    return pl.pallas_call(
        _prep_body,
        out_shape=(
            jax.ShapeDtypeStruct((N_NODES, D), jnp.float32),
            jax.ShapeDtypeStruct((N_NODES,), jnp.float32),
            jax.ShapeDtypeStruct((N_NODES,), jnp.float32),
        ),
    )(node_states, w, a1, a2)


def _sc_body(h_hbm, s1_hbm, s2_hbm, dst_hbm, src_hbm,
             num_out, den_out,
             s1r_v, s2_v, dstA, srcA, dstB, srcB, cdA, csA, cdB, csB,
             w_v, rowsA, rowsB, acc_v, den_v, esemA, esemB, gsemA, gsemB):
    cid = lax.axis_index("c")
    sid = lax.axis_index("s")
    wid = sid * NC + cid
    lo = wid * RPT
    hi = lo + RPT

    pltpu.sync_copy(s1_hbm.at[pl.ds(lo, RPT)], s1r_v)
    pltpu.sync_copy(s2_hbm, s2_v)

    zeros = jnp.zeros((16,), jnp.float32)
    izeros = jnp.zeros((16,), jnp.int32)
    lane0 = lax.iota(jnp.int32, 16) == 0

    def _zero(i, _):
        for j in range(D // 16):
            acc_v[i, pl.ds(j * 16, 16)] = zeros
        return 0
    lax.fori_loop(0, RPT, _zero, 0)

    def _zden(z, _):
        den_v[pl.ds(z * 16, 16)] = zeros
        return 0
    lax.fori_loop(0, (RPT + 16) // 16, _zden, 0)

    def issue_edges(c, dstb, srcb, sem):
        e0 = c * CC
        pltpu.async_copy(dst_hbm.at[pl.ds(e0, CC)], dstb, sem)
        pltpu.async_copy(src_hbm.at[pl.ds(e0, CC)], srcb, sem)

    def wait_edges(dstb, srcb, sem):
        pltpu.make_async_copy(dst_hbm.at[pl.ds(0, CC)], dstb, sem).wait()
        pltpu.make_async_copy(src_hbm.at[pl.ds(0, CC)], srcb, sem).wait()

    def compact(dstb, srcb, cd, cs):
        def _cmp(g, cnt):
            sl = pl.ds(g * 16, 16)
            d16 = dstb[sl]
            s16 = srcb[sl]
            msk = (d16 >= lo) & (d16 < hi)
            plsc.store_compressed(cd.at[pl.ds(cnt, 16)], d16 - lo, mask=msk)
            plsc.store_compressed(cs.at[pl.ds(cnt, 16)], s16, mask=msk)
            return cnt + plsc.all_reduce_population_count(msk)[0]
        m = lax.fori_loop(0, CC // 16, _cmp, jnp.int32(0))
        for p in range(SUB // 16):
            cd[pl.ds(m + p * 16, 16)] = izeros
            cs[pl.ds(m + p * 16, 16)] = izeros
        return m

    def issue_gather(cs, rows, sem, i0):
        pltpu.async_copy(h_hbm.at[cs.at[pl.ds(i0, SUB)]], rows, sem)

    def wait_gather(cs, rows, sem, i0):
        pltpu.make_async_copy(h_hbm.at[cs.at[pl.ds(i0, SUB)]], rows,
                              sem).wait()

    def wcompute(cd, cs, i0, m):
        for g in range(SUB // 16):
            csl = pl.ds(i0 + g * 16, 16)
            dl16 = cd[csl]
            s16 = cs[csl]
            x = (plsc.load_gather(s1r_v, [dl16])
                 + plsc.load_gather(s2_v, [s16]))
            x = jnp.where(x >= 0.0, x, 0.01 * x)
            x = jnp.minimum(jnp.maximum(x, -2.0), 2.0)
            lane = i0 + g * 16 + lax.iota(jnp.int32, 16)
            w_v[pl.ds(g * 16, 16)] = jnp.where(lane < m, jnp.exp(x), 0.0)

    def accum128(cd, rows, i0):
        def _edge(e, dloc):
            dloc_next = cd[pl.ds(i0 + e + 1, 16)][0]
            wv = jnp.full((16,), w_v[pl.ds(e, 16)][0], jnp.float32)
            prods = [rows[e, pl.ds(j * 16, 16)] * wv
                     for j in range(D // 16)]
            plsc.addupdate(den_v.at[pl.ds(dloc, 16)],
                           jnp.where(lane0, wv, 0.0))
            for j in range(D // 16):
                plsc.addupdate(acc_v.at[dloc, pl.ds(j * 16, 16)], prods[j])
            return dloc_next
        lax.fori_loop(0, SUB, _edge, cd[pl.ds(i0, 16)][0])

    def accumulate(cd, cs, rows, sem, m):
        wait_gather(cs, rows, sem, 0)
        wcompute(cd, cs, 0, m)
        accum128(cd, rows, 0)
        nsub = (m + SUB - 1) // SUB

        def _slow(q, _):
            i0 = q * SUB
            issue_gather(cs, rows, sem, i0)
            wait_gather(cs, rows, sem, i0)
            wcompute(cd, cs, i0, m)
            accum128(cd, rows, i0)
            return 0
        lax.fori_loop(1, nsub, _slow, 0)

    issue_edges(0, dstA, srcA, esemA)
    wait_edges(dstA, srcA, esemA)
    mA0 = compact(dstA, srcA, cdA, csA)
    issue_gather(csA, rowsA, gsemA, 0)
    issue_edges(1, dstB, srcB, esemB)

    def _pair(k, mA):
        wait_edges(dstB, srcB, esemB)
        mB = compact(dstB, srcB, cdB, csB)
        issue_gather(csB, rowsB, gsemB, 0)
        issue_edges(2 * k + 2, dstA, srcA, esemA)
        accumulate(cdA, csA, rowsA, gsemA, mA)
        wait_edges(dstA, srcA, esemA)
        mA2 = compact(dstA, srcA, cdA, csA)
        issue_gather(csA, rowsA, gsemA, 0)
        issue_edges(2 * k + 3, dstB, srcB, esemB)
        accumulate(cdB, csB, rowsB, gsemB, mB)
        return mA2

    m_last = lax.fori_loop(0, (N_CHUNKS - 2) // 2, _pair, mA0)

    wait_edges(dstB, srcB, esemB)
    mB = compact(dstB, srcB, cdB, csB)
    issue_gather(csB, rowsB, gsemB, 0)
    accumulate(cdA, csA, rowsA, gsemA, m_last)
    accumulate(cdB, csB, rowsB, gsemB, mB)

    pltpu.sync_copy(acc_v, num_out.at[pl.ds(lo, RPT)])
    pltpu.sync_copy(den_v.at[pl.ds(0, RPT)], den_out.at[pl.ds(lo, RPT)])


def _sc_accum(h, s1p, s2p, dst, src):
    mesh = plsc.VectorSubcoreMesh(core_axis_name="c", subcore_axis_name="s",
                                  num_cores=NC)
    fn = functools.partial(
        pl.kernel,
        mesh=mesh,
        out_type=(
            jax.ShapeDtypeStruct((N_PAD, D), jnp.float32),
            jax.ShapeDtypeStruct((N_PAD,), jnp.float32),
        ),
        scratch_types=[
            pltpu.VMEM((RPT,), jnp.float32),
            pltpu.VMEM((N_PAD,), jnp.float32),
            pltpu.VMEM((CC,), jnp.int32),
            pltpu.VMEM((CC,), jnp.int32),
            pltpu.VMEM((CC,), jnp.int32),
            pltpu.VMEM((CC,), jnp.int32),
            pltpu.VMEM((CCP,), jnp.int32),
            pltpu.VMEM((CCP,), jnp.int32),
            pltpu.VMEM((CCP,), jnp.int32),
            pltpu.VMEM((CCP,), jnp.int32),
            pltpu.VMEM((SUB + 16,), jnp.float32),
            pltpu.VMEM((SUB, D), jnp.float32),
            pltpu.VMEM((SUB, D), jnp.float32),
            pltpu.VMEM((RPT, D), jnp.float32),
            pltpu.VMEM((RPT + 16,), jnp.float32),
            pltpu.SemaphoreType.DMA,
            pltpu.SemaphoreType.DMA,
            pltpu.SemaphoreType.DMA,
            pltpu.SemaphoreType.DMA,
        ],
        compiler_params=pltpu.CompilerParams(needs_layout_passes=False),
    )(_sc_body)
    return fn(h, s1p, s2p, dst, src)


def _fin_body(num_ref, den_ref, out_ref):
    den = den_ref[...]
    den = jnp.where(den > 0.0, den, 1.0)
    out_ref[...] = num_ref[...] / den


def _finish(num, den):
    blk = 1000
    grid = N_NODES // blk
    return pl.pallas_call(
        _fin_body,
        grid=(grid,),
        in_specs=[
            pl.BlockSpec((blk, D), lambda i: (i, 0)),
            pl.BlockSpec((blk, 1), lambda i: (i, 0)),
        ],
        out_specs=pl.BlockSpec((blk, D), lambda i: (i, 0)),
        out_shape=jax.ShapeDtypeStruct((N_NODES, D), jnp.float32),
    )(num, den)


def kernel(node_states, edges, kernel, kernel_attention):
    a = kernel_attention[:, 0]
    a1 = a[:D]
    a2 = a[D:]
    h, s1, s2 = _prep(node_states, kernel, a1, a2)
    s1p = jnp.pad(s1, (0, N_PAD - N_NODES))
    s2p = jnp.pad(s2, (0, N_PAD - N_NODES))
    dst = edges[:, 0].astype(jnp.int32)
    src = edges[:, 1].astype(jnp.int32)
    num, den = _sc_accum(h, s1p, s2p, dst, src)
    return _finish(num, den.reshape(N_PAD, 1))


# speedup vs baseline: 2.0300x; 1.0035x over previous
import functools

import jax
import jax.numpy as jnp
from jax import lax
from jax.experimental import pallas as pl
from jax.experimental.pallas import tpu as pltpu
from jax.experimental.pallas import tpu_sc as plsc

N_NODES = 10000
N_EDGES = 320000
D = 128

NC = 2
NS = 16
NW = NC * NS
N_PAD = 10240
RPT = N_PAD // NW
CC = 3200
N_CHUNKS = N_EDGES // CC
SUB = 128
CCP = CC + SUB + 16


def _prep_body(ns_ref, w_ref, a8_ref, h_ref, s1_ref, s2_ref):
    h = jnp.dot(ns_ref[...], w_ref[...], preferred_element_type=jnp.float32)
    h_ref[...] = h
    s8 = jnp.dot(h, a8_ref[...], preferred_element_type=jnp.float32)
    s1_ref[...] = s8[:, 0]
    s2_ref[...] = s8[:, 1]


def _prep(node_states, w, a8):
    return pl.pallas_call(
        _prep_body,
        out_shape=(
            jax.ShapeDtypeStruct((N_NODES, D), jnp.float32),
            jax.ShapeDtypeStruct((N_NODES,), jnp.float32),
            jax.ShapeDtypeStruct((N_NODES,), jnp.float32),
        ),
    )(node_states, w, a8)


def _sc_body(h_hbm, s1_hbm, s2_hbm, dst_hbm, src_hbm,
             num_out, den_out,
             s1r_v, s2_v, dstA, srcA, dstB, srcB, cdA, csA, cdB, csB,
             w_v, rowsA, rowsB, acc_v, den_v, esemA, esemB, gsemA, gsemB):
    cid = lax.axis_index("c")
    sid = lax.axis_index("s")
    wid = sid * NC + cid
    lo = wid * RPT
    hi = lo + RPT

    pltpu.sync_copy(s1_hbm.at[pl.ds(lo, RPT)], s1r_v)
    pltpu.sync_copy(s2_hbm, s2_v)

    zeros = jnp.zeros((16,), jnp.float32)
    izeros = jnp.zeros((16,), jnp.int32)
    lane0 = lax.iota(jnp.int32, 16) == 0

    def _zero(i, _):
        for j in range(D // 16):
            acc_v[i, pl.ds(j * 16, 16)] = zeros
        return 0
    lax.fori_loop(0, RPT, _zero, 0)

    def _zden(z, _):
        den_v[pl.ds(z * 16, 16)] = zeros
        return 0
    lax.fori_loop(0, (RPT + 16) // 16, _zden, 0)

    def issue_edges(c, dstb, srcb, sem):
        e0 = c * CC
        pltpu.async_copy(dst_hbm.at[pl.ds(e0, CC)], dstb, sem)
        pltpu.async_copy(src_hbm.at[pl.ds(e0, CC)], srcb, sem)

    def wait_edges(dstb, srcb, sem):
        pltpu.make_async_copy(dst_hbm.at[pl.ds(0, CC)], dstb, sem).wait()
        pltpu.make_async_copy(src_hbm.at[pl.ds(0, CC)], srcb, sem).wait()

    def compact(dstb, srcb, cd, cs):
        def _cmp(g, cnt):
            sl = pl.ds(g * 16, 16)
            d16 = dstb[sl]
            s16 = srcb[sl]
            msk = (d16 >= lo) & (d16 < hi)
            plsc.store_compressed(cd.at[pl.ds(cnt, 16)], d16 - lo, mask=msk)
            plsc.store_compressed(cs.at[pl.ds(cnt, 16)], s16, mask=msk)
            return cnt + plsc.all_reduce_population_count(msk)[0]
        m = lax.fori_loop(0, CC // 16, _cmp, jnp.int32(0))

        def _pad(p, _):
            cd[pl.ds(m + p * 16, 16)] = izeros
            cs[pl.ds(m + p * 16, 16)] = izeros
            return 0
        lax.fori_loop(0, SUB // 16, _pad, 0)
        return m

    def issue_gather(cs, rows, sem, i0):
        pltpu.async_copy(h_hbm.at[cs.at[pl.ds(i0, SUB)]], rows, sem)

    def wait_gather(cs, rows, sem, i0):
        pltpu.make_async_copy(h_hbm.at[cs.at[pl.ds(i0, SUB)]], rows,
                              sem).wait()

    def wcompute(cd, cs, i0, m):
        def _wg(g, _):
            csl = pl.ds(i0 + g * 16, 16)
            dl16 = cd[csl]
            s16 = cs[csl]
            x = (plsc.load_gather(s1r_v, [dl16])
                 + plsc.load_gather(s2_v, [s16]))
            x = jnp.where(x >= 0.0, x, 0.01 * x)
            x = jnp.minimum(jnp.maximum(x, -2.0), 2.0)
            lane = i0 + g * 16 + lax.iota(jnp.int32, 16)
            w_v[pl.ds(g * 16, 16)] = jnp.where(lane < m, jnp.exp(x), 0.0)
            return 0
        lax.fori_loop(0, SUB // 16, _wg, 0)

    def accum128(cd, rows, i0, n):
        def _edge(e, dloc):
            dloc_next = cd[pl.ds(i0 + e + 1, 16)][0]
            wv = jnp.full((16,), w_v[pl.ds(e, 16)][0], jnp.float32)
            prods = [rows[e, pl.ds(j * 16, 16)] * wv
                     for j in range(D // 16)]
            plsc.addupdate(den_v.at[pl.ds(dloc, 16)],
                           jnp.where(lane0, wv, 0.0))
            for j in range(D // 16):
                plsc.addupdate(acc_v.at[dloc, pl.ds(j * 16, 16)], prods[j])
            return dloc_next
        lax.fori_loop(0, n, _edge, cd[pl.ds(i0, 16)][0])

    def accumulate(cd, cs, rows, sem, m):
        wait_gather(cs, rows, sem, 0)
        wcompute(cd, cs, 0, m)
        accum128(cd, rows, 0, jnp.minimum(m, SUB))
        nsub = (m + SUB - 1) // SUB

        def _slow(q, _):
            i0 = q * SUB
            issue_gather(cs, rows, sem, i0)
            wait_gather(cs, rows, sem, i0)
            wcompute(cd, cs, i0, m)
            accum128(cd, rows, i0, jnp.minimum(m - i0, SUB))
            return 0
        lax.fori_loop(1, nsub, _slow, 0)

    issue_edges(0, dstA, srcA, esemA)
    wait_edges(dstA, srcA, esemA)
    mA0 = compact(dstA, srcA, cdA, csA)
    issue_gather(csA, rowsA, gsemA, 0)
    issue_edges(1, dstB, srcB, esemB)

    def _pair(k, mA):
        wait_edges(dstB, srcB, esemB)
        mB = compact(dstB, srcB, cdB, csB)
        issue_gather(csB, rowsB, gsemB, 0)
        issue_edges(2 * k + 2, dstA, srcA, esemA)
        accumulate(cdA, csA, rowsA, gsemA, mA)
        wait_edges(dstA, srcA, esemA)
        mA2 = compact(dstA, srcA, cdA, csA)
        issue_gather(csA, rowsA, gsemA, 0)
        issue_edges(2 * k + 3, dstB, srcB, esemB)
        accumulate(cdB, csB, rowsB, gsemB, mB)
        return mA2

    m_last = lax.fori_loop(0, (N_CHUNKS - 2) // 2, _pair, mA0)

    wait_edges(dstB, srcB, esemB)
    mB = compact(dstB, srcB, cdB, csB)
    issue_gather(csB, rowsB, gsemB, 0)
    accumulate(cdA, csA, rowsA, gsemA, m_last)
    accumulate(cdB, csB, rowsB, gsemB, mB)

    pltpu.sync_copy(acc_v, num_out.at[pl.ds(lo, RPT)])
    pltpu.sync_copy(den_v.at[pl.ds(0, RPT)], den_out.at[pl.ds(lo, RPT)])


def _sc_accum(h, s1p, s2p, dst, src):
    mesh = plsc.VectorSubcoreMesh(core_axis_name="c", subcore_axis_name="s",
                                  num_cores=NC)
    fn = functools.partial(
        pl.kernel,
        mesh=mesh,
        out_type=(
            jax.ShapeDtypeStruct((N_PAD, D), jnp.float32),
            jax.ShapeDtypeStruct((N_PAD,), jnp.float32),
        ),
        scratch_types=[
            pltpu.VMEM((RPT,), jnp.float32),
            pltpu.VMEM((N_PAD,), jnp.float32),
            pltpu.VMEM((CC,), jnp.int32),
            pltpu.VMEM((CC,), jnp.int32),
            pltpu.VMEM((CC,), jnp.int32),
            pltpu.VMEM((CC,), jnp.int32),
            pltpu.VMEM((CCP,), jnp.int32),
            pltpu.VMEM((CCP,), jnp.int32),
            pltpu.VMEM((CCP,), jnp.int32),
            pltpu.VMEM((CCP,), jnp.int32),
            pltpu.VMEM((SUB + 16,), jnp.float32),
            pltpu.VMEM((SUB, D), jnp.float32),
            pltpu.VMEM((SUB, D), jnp.float32),
            pltpu.VMEM((RPT, D), jnp.float32),
            pltpu.VMEM((RPT + 16,), jnp.float32),
            pltpu.SemaphoreType.DMA,
            pltpu.SemaphoreType.DMA,
            pltpu.SemaphoreType.DMA,
            pltpu.SemaphoreType.DMA,
        ],
        compiler_params=pltpu.CompilerParams(needs_layout_passes=False),
    )(_sc_body)
    return fn(h, s1p, s2p, dst, src)


def _fin_body(num_ref, den_ref, out_ref):
    den = den_ref[...]
    den = jnp.where(den > 0.0, den, 1.0)
    out_ref[...] = num_ref[...] / den


def _finish(num, den):
    blk = 1000
    grid = N_NODES // blk
    return pl.pallas_call(
        _fin_body,
        grid=(grid,),
        in_specs=[
            pl.BlockSpec((blk, D), lambda i: (i, 0)),
            pl.BlockSpec((blk, 1), lambda i: (i, 0)),
        ],
        out_specs=pl.BlockSpec((blk, D), lambda i: (i, 0)),
        out_shape=jax.ShapeDtypeStruct((N_NODES, D), jnp.float32),
    )(num, den)


def kernel(node_states, edges, kernel, kernel_attention):
    a = kernel_attention[:, 0]
    a8 = jnp.stack([a[:D], a[D:]] + [jnp.zeros((D,), jnp.float32)] * 6,
                   axis=1)
    h, s1, s2 = _prep(node_states, kernel, a8)
    s1p = jnp.pad(s1, (0, N_PAD - N_NODES))
    s2p = jnp.pad(s2, (0, N_PAD - N_NODES))
    dst = edges[:, 0].astype(jnp.int32)
    src = edges[:, 1].astype(jnp.int32)
    num, den = _sc_accum(h, s1p, s2p, dst, src)
    return _finish(num, den.reshape(N_PAD, 1))


# speedup vs baseline: 2.0426x; 1.0062x over previous
import functools

import jax
import jax.numpy as jnp
from jax import lax
from jax.experimental import pallas as pl
from jax.experimental.pallas import tpu as pltpu
from jax.experimental.pallas import tpu_sc as plsc

N_NODES = 10000
N_EDGES = 320000
D = 128

NC = 2
NS = 16
NW = NC * NS
N_PAD = 10240
RPT = N_PAD // NW
CC = 3200
N_CHUNKS = N_EDGES // CC
SUB = 128
GK = 8
GR = SUB // GK
CCP = CC + SUB + 16


def _prep_body(ns_ref, w_ref, a8_ref, h_ref, s1_ref, s2_ref):
    h = jnp.dot(ns_ref[...], w_ref[...], preferred_element_type=jnp.float32)
    h_ref[...] = h
    s8 = jnp.dot(h, a8_ref[...], preferred_element_type=jnp.float32)
    s1_ref[...] = s8[:, 0]
    s2_ref[...] = s8[:, 1]


def _prep(node_states, w, a8):
    return pl.pallas_call(
        _prep_body,
        out_shape=(
            jax.ShapeDtypeStruct((N_NODES, D), jnp.float32),
            jax.ShapeDtypeStruct((N_NODES,), jnp.float32),
            jax.ShapeDtypeStruct((N_NODES,), jnp.float32),
        ),
    )(node_states, w, a8)


def _sc_body(h_hbm, s1_hbm, s2_hbm, dst_hbm, src_hbm,
             num_out, den_out,
             s1r_v, s2_v, dstA, srcA, dstB, srcB, cdA, csA, cdB, csB,
             w_v, rowsA, rowsB, acc_v, den_v, esemA, esemB, gsemA, gsemB):
    cid = lax.axis_index("c")
    sid = lax.axis_index("s")
    wid = sid * NC + cid
    lo = wid * RPT
    hi = lo + RPT

    pltpu.sync_copy(s1_hbm.at[pl.ds(lo, RPT)], s1r_v)
    pltpu.sync_copy(s2_hbm, s2_v)

    zeros = jnp.zeros((16,), jnp.float32)
    izeros = jnp.zeros((16,), jnp.int32)
    lane0 = lax.iota(jnp.int32, 16) == 0

    def _zero(i, _):
        for j in range(D // 16):
            acc_v[i, pl.ds(j * 16, 16)] = zeros
        return 0
    lax.fori_loop(0, RPT, _zero, 0)

    def _zden(z, _):
        den_v[pl.ds(z * 16, 16)] = zeros
        return 0
    lax.fori_loop(0, (RPT + 16) // 16, _zden, 0)

    def issue_edges(c, dstb, srcb, sem):
        e0 = c * CC
        pltpu.async_copy(dst_hbm.at[pl.ds(e0, CC)], dstb, sem)
        pltpu.async_copy(src_hbm.at[pl.ds(e0, CC)], srcb, sem)

    def wait_edges(dstb, srcb, sem):
        pltpu.make_async_copy(dst_hbm.at[pl.ds(0, CC)], dstb, sem).wait()
        pltpu.make_async_copy(src_hbm.at[pl.ds(0, CC)], srcb, sem).wait()

    def compact(dstb, srcb, cd, cs):
        def _cmp(g, cnt):
            sl = pl.ds(g * 16, 16)
            d16 = dstb[sl]
            s16 = srcb[sl]
            msk = (d16 >= lo) & (d16 < hi)
            plsc.store_compressed(cd.at[pl.ds(cnt, 16)], d16 - lo, mask=msk)
            plsc.store_compressed(cs.at[pl.ds(cnt, 16)], s16, mask=msk)
            return cnt + plsc.all_reduce_population_count(msk)[0]
        m = lax.fori_loop(0, CC // 16, _cmp, jnp.int32(0))

        def _pad(p, _):
            cd[pl.ds(m + p * 16, 16)] = izeros
            cs[pl.ds(m + p * 16, 16)] = izeros
            return 0
        lax.fori_loop(0, SUB // 16, _pad, 0)
        return m

    def issue_gather(cs, rows, sem, i0):
        for j in range(GK):
            pltpu.async_copy(h_hbm.at[cs.at[pl.ds(i0 + j * GR, GR)]],
                             rows.at[pl.ds(j * GR, GR)], sem)

    def wait_gather(cs, rows, sem, i0):
        for j in range(GK):
            pltpu.make_async_copy(h_hbm.at[cs.at[pl.ds(i0 + j * GR, GR)]],
                                  rows.at[pl.ds(j * GR, GR)], sem).wait()

    def wcompute(cd, cs, i0, m):
        def _wg(g, _):
            csl = pl.ds(i0 + g * 16, 16)
            dl16 = cd[csl]
            s16 = cs[csl]
            x = (plsc.load_gather(s1r_v, [dl16])
                 + plsc.load_gather(s2_v, [s16]))
            x = jnp.where(x >= 0.0, x, 0.01 * x)
            x = jnp.minimum(jnp.maximum(x, -2.0), 2.0)
            lane = i0 + g * 16 + lax.iota(jnp.int32, 16)
            w_v[pl.ds(g * 16, 16)] = jnp.where(lane < m, jnp.exp(x), 0.0)
            return 0
        lax.fori_loop(0, SUB // 16, _wg, 0)

    def accum128(cd, rows, i0, n):
        def _edge(e, dloc):
            dloc_next = cd[pl.ds(i0 + e + 1, 16)][0]
            wv = jnp.full((16,), w_v[pl.ds(e, 16)][0], jnp.float32)
            prods = [rows[e, pl.ds(j * 16, 16)] * wv
                     for j in range(D // 16)]
            plsc.addupdate(den_v.at[pl.ds(dloc, 16)],
                           jnp.where(lane0, wv, 0.0))
            for j in range(D // 16):
                plsc.addupdate(acc_v.at[dloc, pl.ds(j * 16, 16)], prods[j])
            return dloc_next
        lax.fori_loop(0, n, _edge, cd[pl.ds(i0, 16)][0])

    def accumulate(cd, cs, rows, sem, m):
        wait_gather(cs, rows, sem, 0)
        wcompute(cd, cs, 0, m)
        accum128(cd, rows, 0, jnp.minimum(m, SUB))
        nsub = (m + SUB - 1) // SUB

        def _slow(q, _):
            i0 = q * SUB
            issue_gather(cs, rows, sem, i0)
            wait_gather(cs, rows, sem, i0)
            wcompute(cd, cs, i0, m)
            accum128(cd, rows, i0, jnp.minimum(m - i0, SUB))
            return 0
        lax.fori_loop(1, nsub, _slow, 0)

    issue_edges(0, dstA, srcA, esemA)
    wait_edges(dstA, srcA, esemA)
    mA0 = compact(dstA, srcA, cdA, csA)
    issue_gather(csA, rowsA, gsemA, 0)
    issue_edges(1, dstB, srcB, esemB)

    def _pair(k, mA):
        wait_edges(dstB, srcB, esemB)
        mB = compact(dstB, srcB, cdB, csB)
        issue_gather(csB, rowsB, gsemB, 0)
        issue_edges(2 * k + 2, dstA, srcA, esemA)
        accumulate(cdA, csA, rowsA, gsemA, mA)
        wait_edges(dstA, srcA, esemA)
        mA2 = compact(dstA, srcA, cdA, csA)
        issue_gather(csA, rowsA, gsemA, 0)
        issue_edges(2 * k + 3, dstB, srcB, esemB)
        accumulate(cdB, csB, rowsB, gsemB, mB)
        return mA2

    m_last = lax.fori_loop(0, (N_CHUNKS - 2) // 2, _pair, mA0)

    wait_edges(dstB, srcB, esemB)
    mB = compact(dstB, srcB, cdB, csB)
    issue_gather(csB, rowsB, gsemB, 0)
    accumulate(cdA, csA, rowsA, gsemA, m_last)
    accumulate(cdB, csB, rowsB, gsemB, mB)

    pltpu.sync_copy(acc_v, num_out.at[pl.ds(lo, RPT)])
    pltpu.sync_copy(den_v.at[pl.ds(0, RPT)], den_out.at[pl.ds(lo, RPT)])


def _sc_accum(h, s1p, s2p, dst, src):
    mesh = plsc.VectorSubcoreMesh(core_axis_name="c", subcore_axis_name="s",
                                  num_cores=NC)
    fn = functools.partial(
        pl.kernel,
        mesh=mesh,
        out_type=(
            jax.ShapeDtypeStruct((N_PAD, D), jnp.float32),
            jax.ShapeDtypeStruct((N_PAD,), jnp.float32),
        ),
        scratch_types=[
            pltpu.VMEM((RPT,), jnp.float32),
            pltpu.VMEM((N_PAD,), jnp.float32),
            pltpu.VMEM((CC,), jnp.int32),
            pltpu.VMEM((CC,), jnp.int32),
            pltpu.VMEM((CC,), jnp.int32),
            pltpu.VMEM((CC,), jnp.int32),
            pltpu.VMEM((CCP,), jnp.int32),
            pltpu.VMEM((CCP,), jnp.int32),
            pltpu.VMEM((CCP,), jnp.int32),
            pltpu.VMEM((CCP,), jnp.int32),
            pltpu.VMEM((SUB + 16,), jnp.float32),
            pltpu.VMEM((SUB, D), jnp.float32),
            pltpu.VMEM((SUB, D), jnp.float32),
            pltpu.VMEM((RPT, D), jnp.float32),
            pltpu.VMEM((RPT + 16,), jnp.float32),
            pltpu.SemaphoreType.DMA,
            pltpu.SemaphoreType.DMA,
            pltpu.SemaphoreType.DMA,
            pltpu.SemaphoreType.DMA,
        ],
        compiler_params=pltpu.CompilerParams(needs_layout_passes=False),
    )(_sc_body)
    return fn(h, s1p, s2p, dst, src)


def _fin_body(num_ref, den_ref, out_ref):
    den = den_ref[...]
    den = jnp.where(den > 0.0, den, 1.0)
    out_ref[...] = num_ref[...] / den


def _finish(num, den):
    blk = 1000
    grid = N_NODES // blk
    return pl.pallas_call(
        _fin_body,
        grid=(grid,),
        in_specs=[
            pl.BlockSpec((blk, D), lambda i: (i, 0)),
            pl.BlockSpec((blk, 1), lambda i: (i, 0)),
        ],
        out_specs=pl.BlockSpec((blk, D), lambda i: (i, 0)),
        out_shape=jax.ShapeDtypeStruct((N_NODES, D), jnp.float32),
    )(num, den)


def kernel(node_states, edges, kernel, kernel_attention):
    a = kernel_attention[:, 0]
    a8 = jnp.stack([a[:D], a[D:]] + [jnp.zeros((D,), jnp.float32)] * 6,
                   axis=1)
    h, s1, s2 = _prep(node_states, kernel, a8)
    s1p = jnp.pad(s1, (0, N_PAD - N_NODES))
    s2p = jnp.pad(s2, (0, N_PAD - N_NODES))
    dst = edges[:, 0].astype(jnp.int32)
    src = edges[:, 1].astype(jnp.int32)
    num, den = _sc_accum(h, s1p, s2p, dst, src)
    return _finish(num, den.reshape(N_PAD, 1))


# speedup vs baseline: 14.5709x; 7.1334x over previous
import functools

import jax
import jax.numpy as jnp
import numpy as np
from jax import lax
from jax.experimental import pallas as pl
from jax.experimental.pallas import tpu as pltpu
from jax.experimental.pallas import tpu_sc as plsc

N_NODES = 10000
N_EDGES = 320000
D = 128

NC = 2
NS = 16
NW = NC * NS
N_PAD = 10240
RPT = N_PAD // NW
CC = 3200
N_CHUNKS = N_EDGES // CC
SUB = 128
GK = 8
GR = SUB // GK
CCP = CC + SUB + 16
NPAIR = N_NODES // 2


def _prep_body(ns_ref, w_ref, wp_ref, a8_ref, hb_ref, s1_ref, s2_ref):
    ns = ns_ref[...]
    hp = jnp.dot(ns, wp_ref[...], preferred_element_type=jnp.float32)
    hb_ref[...] = hp.astype(jnp.bfloat16)
    wa = jnp.dot(w_ref[...], a8_ref[...], preferred_element_type=jnp.float32)
    s8 = jnp.dot(ns, wa, preferred_element_type=jnp.float32)
    s1_ref[...] = s8[:, 0]
    s2_ref[...] = s8[:, 1]


def _prep(node_states, w, wp, a8):
    return pl.pallas_call(
        _prep_body,
        out_shape=(
            jax.ShapeDtypeStruct((N_NODES, D), jnp.bfloat16),
            jax.ShapeDtypeStruct((N_NODES,), jnp.float32),
            jax.ShapeDtypeStruct((N_NODES,), jnp.float32),
        ),
    )(node_states, w, wp, a8)


def _sc_body(h_hbm, s1_hbm, s2_hbm, dst_hbm, src_hbm,
             num_out, den_out,
             s1r_v, s2_v, dstc_v, srcc_v, cd_v, ch_v,
             w_v, rows_v, acc_v, den_v, h_sh, esem, gsem):
    cid = lax.axis_index("c")
    sid = lax.axis_index("s")
    wid = sid * NC + cid
    lo = wid * RPT
    hi = lo + RPT

    pltpu.sync_copy(s1_hbm.at[pl.ds(lo, RPT)], s1r_v)
    pltpu.sync_copy(s2_hbm, s2_v)

    @pl.when(sid == 0)
    def _():
        pltpu.sync_copy(h_hbm, h_sh)

    zeros = jnp.zeros((16,), jnp.float32)
    izeros = jnp.zeros((16,), jnp.int32)
    lane0 = lax.iota(jnp.int32, 16) == 0

    def _zero(i, _):
        for j in range(D // 16):
            acc_v[i, pl.ds(j * 16, 16)] = zeros
        return 0
    lax.fori_loop(0, RPT, _zero, 0)

    def _zden(z, _):
        den_v[pl.ds(z * 16, 16)] = zeros
        return 0
    lax.fori_loop(0, (RPT + 16) // 16, _zden, 0)

    plsc.subcore_barrier()

    def issue_edges(c):
        e0 = c * CC
        pltpu.async_copy(dst_hbm.at[pl.ds(e0, CC)], dstc_v, esem)
        pltpu.async_copy(src_hbm.at[pl.ds(e0, CC)], srcc_v, esem)

    def wait_edges():
        pltpu.make_async_copy(dst_hbm.at[pl.ds(0, CC)], dstc_v, esem).wait()
        pltpu.make_async_copy(src_hbm.at[pl.ds(0, CC)], srcc_v, esem).wait()

    def compact():
        def _cmp(g, cnt):
            sl = pl.ds(g * 16, 16)
            d16 = dstc_v[sl]
            s16 = srcc_v[sl]
            msk = (d16 >= lo) & (d16 < hi)
            packed = (d16 - lo) | ((s16 & 1) << 12)
            plsc.store_compressed(cd_v.at[pl.ds(cnt, 16)], packed, mask=msk)
            plsc.store_compressed(ch_v.at[pl.ds(cnt, 16)],
                                  lax.shift_right_logical(s16, 1), mask=msk)
            return cnt + plsc.all_reduce_population_count(msk)[0]
        m = lax.fori_loop(0, CC // 16, _cmp, jnp.int32(0))

        def _pad(p, _):
            cd_v[pl.ds(m + p * 16, 16)] = izeros
            ch_v[pl.ds(m + p * 16, 16)] = izeros
            return 0
        lax.fori_loop(0, SUB // 16, _pad, 0)
        return m

    def issue_gather(i0):
        for j in range(GK):
            pltpu.async_copy(h_sh.at[ch_v.at[pl.ds(i0 + j * GR, GR)]],
                             rows_v.at[pl.ds(j * GR, GR)], gsem)

    def wait_gather(i0):
        for j in range(GK):
            pltpu.make_async_copy(h_sh.at[ch_v.at[pl.ds(i0 + j * GR, GR)]],
                                  rows_v.at[pl.ds(j * GR, GR)], gsem).wait()

    def wcompute(i0, m):
        def _wg(g, _):
            csl = pl.ds(i0 + g * 16, 16)
            raw16 = cd_v[csl]
            dl16 = raw16 & 4095
            s16 = (ch_v[csl] << 1) | (lax.shift_right_logical(raw16, 12) & 1)
            x = (plsc.load_gather(s1r_v, [dl16])
                 + plsc.load_gather(s2_v, [s16]))
            x = jnp.where(x >= 0.0, x, 0.01 * x)
            x = jnp.minimum(jnp.maximum(x, -2.0), 2.0)
            lane = i0 + g * 16 + lax.iota(jnp.int32, 16)
            w_v[pl.ds(g * 16, 16)] = jnp.where(lane < m, jnp.exp(x), 0.0)
            return 0
        lax.fori_loop(0, SUB // 16, _wg, 0)

    def accum(i0, n):
        def _edge(e, raw):
            raw_next = cd_v[pl.ds(i0 + e + 1, 16)][0]
            dloc = raw & 4095
            coff = lax.shift_right_logical(raw, 12) * 64
            wv = jnp.full((16,), w_v[pl.ds(e, 16)][0], jnp.float32)
            prods = []
            for j in range(D // 32):
                x32 = plsc.bitcast(rows_v[e, pl.ds(coff + j * 16, 16)],
                                   jnp.bfloat16)
                a, b = plsc.unpack(x32, format=plsc.PackFormat.INTERLEAVED)
                prods += [a * wv, b * wv]
            plsc.addupdate(den_v.at[pl.ds(dloc, 16)],
                           jnp.where(lane0, wv, 0.0))
            for j in range(D // 16):
                plsc.addupdate(acc_v.at[dloc, pl.ds(j * 16, 16)], prods[j])
            return raw_next
        lax.fori_loop(0, n, _edge, cd_v[pl.ds(i0, 16)][0])

    issue_edges(0)

    def _chunk(c, _):
        wait_edges()
        m = compact()

        @pl.when(c + 1 < N_CHUNKS)
        def _():
            issue_edges(c + 1)
        issue_gather(0)
        wcompute(0, m)
        wait_gather(0)
        accum(0, jnp.minimum(m, SUB))

        nsub = (m + SUB - 1) // SUB

        def _slow(q, _):
            i0 = q * SUB
            issue_gather(i0)
            wcompute(i0, m)
            wait_gather(i0)
            accum(i0, jnp.minimum(m - i0, SUB))
            return 0
        lax.fori_loop(1, nsub, _slow, 0)
        return 0

    lax.fori_loop(0, N_CHUNKS, _chunk, 0)

    pltpu.sync_copy(acc_v, num_out.at[pl.ds(lo, RPT)])
    pltpu.sync_copy(den_v.at[pl.ds(0, RPT)], den_out.at[pl.ds(lo, RPT)])


def _sc_accum(h, s1p, s2p, dst, src):
    mesh = plsc.VectorSubcoreMesh(core_axis_name="c", subcore_axis_name="s",
                                  num_cores=NC)
    fn = functools.partial(
        pl.kernel,
        mesh=mesh,
        out_type=(
            jax.ShapeDtypeStruct((N_PAD, D), jnp.float32),
            jax.ShapeDtypeStruct((N_PAD,), jnp.float32),
        ),
        scratch_types=[
            pltpu.VMEM((RPT,), jnp.float32),
            pltpu.VMEM((N_PAD,), jnp.float32),
            pltpu.VMEM((CC,), jnp.int32),
            pltpu.VMEM((CC,), jnp.int32),
            pltpu.VMEM((CCP,), jnp.int32),
            pltpu.VMEM((CCP,), jnp.int32),
            pltpu.VMEM((SUB + 16,), jnp.float32),
            pltpu.VMEM((SUB, D), jnp.int32),
            pltpu.VMEM((RPT, D), jnp.float32),
            pltpu.VMEM((RPT + 16,), jnp.float32),
            pltpu.VMEM_SHARED((NPAIR, D), jnp.int32),
            pltpu.SemaphoreType.DMA,
            pltpu.SemaphoreType.DMA,
        ],
        compiler_params=pltpu.CompilerParams(needs_layout_passes=False),
    )(_sc_body)
    return fn(h, s1p, s2p, dst, src)


def _fin_body(num_ref, den_ref, out_ref):
    den = den_ref[...]
    den = jnp.where(den > 0.0, den, 1.0)
    out_ref[...] = num_ref[...] / den


def _finish(num, den):
    blk = 1000
    grid = N_NODES // blk
    return pl.pallas_call(
        _fin_body,
        grid=(grid,),
        in_specs=[
            pl.BlockSpec((blk, D), lambda i: (i, 0)),
            pl.BlockSpec((blk, 1), lambda i: (i, 0)),
        ],
        out_specs=pl.BlockSpec((blk, D), lambda i: (i, 0)),
        out_shape=jax.ShapeDtypeStruct((N_NODES, D), jnp.float32),
    )(num, den)


def kernel(node_states, edges, kernel, kernel_attention):
    a = kernel_attention[:, 0]
    a8 = jnp.stack([a[:D], a[D:]] + [jnp.zeros((D,), jnp.float32)] * 6,
                   axis=1)
    g = np.empty((D,), np.int32)
    for j in range(D // 32):
        for t in range(16):
            g[32 * j + 2 * t] = 32 * j + t
            g[32 * j + 2 * t + 1] = 32 * j + 16 + t
    wp = kernel[:, g]
    h, s1, s2 = _prep(node_states, kernel, wp, a8)
    h = lax.bitcast_convert_type(h.reshape(N_NODES, D // 2, 2), jnp.int32)
    h = h.reshape(NPAIR, D)
    s1p = jnp.pad(s1, (0, N_PAD - N_NODES))
    s2p = jnp.pad(s2, (0, N_PAD - N_NODES))
    dst = edges[:, 0].astype(jnp.int32)
    src = edges[:, 1].astype(jnp.int32)
    num, den = _sc_accum(h, s1p, s2p, dst, src)
    return _finish(num, den.reshape(N_PAD, 1))


# speedup vs baseline: 16.0146x; 1.0991x over previous
import functools

import jax
import jax.numpy as jnp
import numpy as np
from jax import lax
from jax.experimental import pallas as pl
from jax.experimental.pallas import tpu as pltpu
from jax.experimental.pallas import tpu_sc as plsc

N_NODES = 10000
N_EDGES = 320000
D = 128

NC = 2
NS = 16
NW = NC * NS
N_PAD = 10240
RPT = N_PAD // NW
CC = 3200
N_CHUNKS = N_EDGES // CC
SUB = 128
GK = 8
GR = SUB // GK
CCP = CC + SUB + 16
NPAIR = N_NODES // 2


def _prep_body(ns_ref, w_ref, wp_ref, a8_ref, hb_ref, s1_ref, s2_ref):
    ns = ns_ref[...]
    hp = jnp.dot(ns, wp_ref[...], preferred_element_type=jnp.float32)
    hb_ref[...] = hp.astype(jnp.bfloat16)
    wa = jnp.dot(w_ref[...], a8_ref[...], preferred_element_type=jnp.float32)
    s8 = jnp.dot(ns, wa, preferred_element_type=jnp.float32)
    s1_ref[...] = s8[:, 0]
    s2_ref[...] = s8[:, 1]


def _prep(node_states, w, wp, a8):
    return pl.pallas_call(
        _prep_body,
        out_shape=(
            jax.ShapeDtypeStruct((N_NODES, D), jnp.bfloat16),
            jax.ShapeDtypeStruct((N_NODES,), jnp.float32),
            jax.ShapeDtypeStruct((N_NODES,), jnp.float32),
        ),
    )(node_states, w, wp, a8)


def _sc_body(h_hbm, s1_hbm, s2_hbm, dst_hbm, src_hbm,
             num_out, den_out,
             s1r_v, s2_v, dstc_v, srcc_v, cd_v, ch_v,
             w_v, rows_v, acc_v, den_v, h_sh, esem, gsem):
    cid = lax.axis_index("c")
    sid = lax.axis_index("s")
    wid = sid * NC + cid
    lo = wid * RPT
    hi = lo + RPT

    pltpu.sync_copy(s1_hbm.at[pl.ds(lo, RPT)], s1r_v)
    pltpu.sync_copy(s2_hbm, s2_v)

    @pl.when(sid == 0)
    def _():
        pltpu.sync_copy(h_hbm, h_sh)

    zeros = jnp.zeros((16,), jnp.float32)
    izeros = jnp.zeros((16,), jnp.int32)
    lane0 = lax.iota(jnp.int32, 16) == 0

    def _zero(i, _):
        for j in range(D // 16):
            acc_v[i, pl.ds(j * 16, 16)] = zeros
        return 0
    lax.fori_loop(0, RPT, _zero, 0)

    def _zden(z, _):
        den_v[pl.ds(z * 16, 16)] = zeros
        return 0
    lax.fori_loop(0, (RPT + 16) // 16, _zden, 0)

    plsc.subcore_barrier()

    def issue_edges(c):
        e0 = c * CC
        pltpu.async_copy(dst_hbm.at[pl.ds(e0, CC)], dstc_v, esem)
        pltpu.async_copy(src_hbm.at[pl.ds(e0, CC)], srcc_v, esem)

    def wait_edges():
        pltpu.make_async_copy(dst_hbm.at[pl.ds(0, CC)], dstc_v, esem).wait()
        pltpu.make_async_copy(src_hbm.at[pl.ds(0, CC)], srcc_v, esem).wait()

    def compact():
        def _cmp(g, cnt):
            sl = pl.ds(g * 16, 16)
            d16 = dstc_v[sl]
            s16 = srcc_v[sl]
            msk = (d16 >= lo) & (d16 < hi)
            packed = (d16 - lo) | ((s16 & 1) << 12)
            plsc.store_compressed(cd_v.at[pl.ds(cnt, 16)], packed, mask=msk)
            plsc.store_compressed(ch_v.at[pl.ds(cnt, 16)],
                                  lax.shift_right_logical(s16, 1), mask=msk)
            return cnt + plsc.all_reduce_population_count(msk)[0]
        m = lax.fori_loop(0, CC // 16, _cmp, jnp.int32(0), unroll=4)

        def _pad(p, _):
            cd_v[pl.ds(m + p * 16, 16)] = izeros
            ch_v[pl.ds(m + p * 16, 16)] = izeros
            return 0
        lax.fori_loop(0, SUB // 16, _pad, 0)
        return m

    def issue_gather(i0):
        for j in range(GK):
            pltpu.async_copy(h_sh.at[ch_v.at[pl.ds(i0 + j * GR, GR)]],
                             rows_v.at[pl.ds(j * GR, GR)], gsem)

    def wait_gather(i0):
        for j in range(GK):
            pltpu.make_async_copy(h_sh.at[ch_v.at[pl.ds(i0 + j * GR, GR)]],
                                  rows_v.at[pl.ds(j * GR, GR)], gsem).wait()

    def wcompute(i0, m):
        def _wg(g, _):
            csl = pl.ds(i0 + g * 16, 16)
            raw16 = cd_v[csl]
            dl16 = raw16 & 4095
            s16 = (ch_v[csl] << 1) | (lax.shift_right_logical(raw16, 12) & 1)
            x = (plsc.load_gather(s1r_v, [dl16])
                 + plsc.load_gather(s2_v, [s16]))
            x = jnp.where(x >= 0.0, x, 0.01 * x)
            x = jnp.minimum(jnp.maximum(x, -2.0), 2.0)
            lane = i0 + g * 16 + lax.iota(jnp.int32, 16)
            w_v[pl.ds(g * 16, 16)] = jnp.where(lane < m, jnp.exp(x), 0.0)
            return 0
        lax.fori_loop(0, SUB // 16, _wg, 0)

    def accum(i0, n):
        def _edge(e, raw):
            raw_next = cd_v[pl.ds(i0 + e + 1, 16)][0]
            dloc = raw & 4095
            coff = lax.shift_right_logical(raw, 12) * 64
            wv = jnp.full((16,), w_v[pl.ds(e, 16)][0], jnp.float32)
            prods = []
            for j in range(D // 32):
                x32 = plsc.bitcast(rows_v[e, pl.ds(coff + j * 16, 16)],
                                   jnp.bfloat16)
                a, b = plsc.unpack(x32, format=plsc.PackFormat.INTERLEAVED)
                prods += [a * wv, b * wv]
            plsc.addupdate(den_v.at[pl.ds(dloc, 16)],
                           jnp.where(lane0, wv, 0.0))
            for j in range(D // 16):
                plsc.addupdate(acc_v.at[dloc, pl.ds(j * 16, 16)], prods[j])
            return raw_next
        lax.fori_loop(0, n, _edge, cd_v[pl.ds(i0, 16)][0])

    issue_edges(0)

    def _chunk(c, _):
        wait_edges()
        m = compact()

        @pl.when(c + 1 < N_CHUNKS)
        def _():
            issue_edges(c + 1)
        issue_gather(0)
        wcompute(0, m)
        wait_gather(0)
        accum(0, jnp.minimum(m, SUB))

        nsub = (m + SUB - 1) // SUB

        def _slow(q, _):
            i0 = q * SUB
            issue_gather(i0)
            wcompute(i0, m)
            wait_gather(i0)
            accum(i0, jnp.minimum(m - i0, SUB))
            return 0
        lax.fori_loop(1, nsub, _slow, 0)
        return 0

    lax.fori_loop(0, N_CHUNKS, _chunk, 0)

    pltpu.sync_copy(acc_v, num_out.at[pl.ds(lo, RPT)])
    pltpu.sync_copy(den_v.at[pl.ds(0, RPT)], den_out.at[pl.ds(lo, RPT)])


def _sc_accum(h, s1p, s2p, dst, src):
    mesh = plsc.VectorSubcoreMesh(core_axis_name="c", subcore_axis_name="s",
                                  num_cores=NC)
    fn = functools.partial(
        pl.kernel,
        mesh=mesh,
        out_type=(
            jax.ShapeDtypeStruct((N_PAD, D), jnp.float32),
            jax.ShapeDtypeStruct((N_PAD,), jnp.float32),
        ),
        scratch_types=[
            pltpu.VMEM((RPT,), jnp.float32),
            pltpu.VMEM((N_PAD,), jnp.float32),
            pltpu.VMEM((CC,), jnp.int32),
            pltpu.VMEM((CC,), jnp.int32),
            pltpu.VMEM((CCP,), jnp.int32),
            pltpu.VMEM((CCP,), jnp.int32),
            pltpu.VMEM((SUB + 16,), jnp.float32),
            pltpu.VMEM((SUB, D), jnp.int32),
            pltpu.VMEM((RPT, D), jnp.float32),
            pltpu.VMEM((RPT + 16,), jnp.float32),
            pltpu.VMEM_SHARED((NPAIR, D), jnp.int32),
            pltpu.SemaphoreType.DMA,
            pltpu.SemaphoreType.DMA,
        ],
        compiler_params=pltpu.CompilerParams(needs_layout_passes=False),
    )(_sc_body)
    return fn(h, s1p, s2p, dst, src)


def _fin_body(num_ref, den_ref, out_ref):
    den = den_ref[...]
    den = jnp.where(den > 0.0, den, 1.0)
    out_ref[...] = num_ref[...] / den


def _finish(num, den):
    blk = 1000
    grid = N_NODES // blk
    return pl.pallas_call(
        _fin_body,
        grid=(grid,),
        in_specs=[
            pl.BlockSpec((blk, D), lambda i: (i, 0)),
            pl.BlockSpec((blk, 1), lambda i: (i, 0)),
        ],
        out_specs=pl.BlockSpec((blk, D), lambda i: (i, 0)),
        out_shape=jax.ShapeDtypeStruct((N_NODES, D), jnp.float32),
    )(num, den)


def kernel(node_states, edges, kernel, kernel_attention):
    a = kernel_attention[:, 0]
    a8 = jnp.stack([a[:D], a[D:]] + [jnp.zeros((D,), jnp.float32)] * 6,
                   axis=1)
    g = np.empty((D,), np.int32)
    for j in range(D // 32):
        for t in range(16):
            g[32 * j + 2 * t] = 32 * j + t
            g[32 * j + 2 * t + 1] = 32 * j + 16 + t
    wp = kernel[:, g]
    h, s1, s2 = _prep(node_states, kernel, wp, a8)
    h = lax.bitcast_convert_type(h.reshape(N_NODES, D // 2, 2), jnp.int32)
    h = h.reshape(NPAIR, D)
    s1p = jnp.pad(s1, (0, N_PAD - N_NODES))
    s2p = jnp.pad(s2, (0, N_PAD - N_NODES))
    dst = edges[:, 0].astype(jnp.int32)
    src = edges[:, 1].astype(jnp.int32)
    num, den = _sc_accum(h, s1p, s2p, dst, src)
    return _finish(num, den.reshape(N_PAD, 1))


# speedup vs baseline: 16.0603x; 1.0029x over previous
import functools

import jax
import jax.numpy as jnp
import numpy as np
from jax import lax
from jax.experimental import pallas as pl
from jax.experimental.pallas import tpu as pltpu
from jax.experimental.pallas import tpu_sc as plsc

N_NODES = 10000
N_EDGES = 320000
D = 128

NC = 2
NS = 16
NW = NC * NS
N_PAD = 10240
RPT = N_PAD // NW
CC = 3200
N_CHUNKS = N_EDGES // CC
SUB = 128
GK = 8
GR = SUB // GK
CCP = CC + SUB + 16
NPAIR = N_NODES // 2


def _prep_body(ns_ref, w_ref, wp_ref, a8_ref, hb_ref, s1_ref, s2_ref):
    ns = ns_ref[...]
    hp = jnp.dot(ns, wp_ref[...], preferred_element_type=jnp.float32)
    hb_ref[...] = hp.astype(jnp.bfloat16)
    wa = jnp.dot(w_ref[...], a8_ref[...], preferred_element_type=jnp.float32)
    s8 = jnp.dot(ns, wa, preferred_element_type=jnp.float32)
    s1_ref[...] = s8[:, 0]
    s2_ref[...] = s8[:, 1]


def _prep(node_states, w, wp, a8):
    return pl.pallas_call(
        _prep_body,
        out_shape=(
            jax.ShapeDtypeStruct((N_NODES, D), jnp.bfloat16),
            jax.ShapeDtypeStruct((N_NODES,), jnp.float32),
            jax.ShapeDtypeStruct((N_NODES,), jnp.float32),
        ),
    )(node_states, w, wp, a8)


def _sc_body(h_hbm, s1_hbm, s2_hbm, dst_hbm, src_hbm,
             num_out, den_out,
             s1r_v, s2_v, dstc_v, srcc_v, cd_v, ch_v,
             w_v, rows_v, acc_v, den_v, h_sh, esem, gsem):
    cid = lax.axis_index("c")
    sid = lax.axis_index("s")
    wid = sid * NC + cid
    lo = wid * RPT
    hi = lo + RPT

    pltpu.sync_copy(s1_hbm.at[pl.ds(lo, RPT)], s1r_v)
    pltpu.sync_copy(s2_hbm, s2_v)

    @pl.when(sid == 0)
    def _():
        pltpu.sync_copy(h_hbm, h_sh)

    zeros = jnp.zeros((16,), jnp.float32)
    izeros = jnp.zeros((16,), jnp.int32)
    lane0 = lax.iota(jnp.int32, 16) == 0

    def _zero(i, _):
        for j in range(D // 16):
            acc_v[i, pl.ds(j * 16, 16)] = zeros
        return 0
    lax.fori_loop(0, RPT, _zero, 0)

    def _zden(z, _):
        den_v[pl.ds(z * 16, 16)] = zeros
        return 0
    lax.fori_loop(0, (RPT + 16) // 16, _zden, 0)

    plsc.subcore_barrier()

    def issue_edges(c):
        e0 = c * CC
        pltpu.async_copy(dst_hbm.at[pl.ds(e0, CC)], dstc_v, esem)
        pltpu.async_copy(src_hbm.at[pl.ds(e0, CC)], srcc_v, esem)

    def wait_edges():
        pltpu.make_async_copy(dst_hbm.at[pl.ds(0, CC)], dstc_v, esem).wait()
        pltpu.make_async_copy(src_hbm.at[pl.ds(0, CC)], srcc_v, esem).wait()

    def compact():
        def _cmp(g, cnt):
            sl = pl.ds(g * 16, 16)
            d16 = dstc_v[sl]
            s16 = srcc_v[sl]
            msk = (d16 >= lo) & (d16 < hi)
            packed = (d16 - lo) | ((s16 & 1) << 12)
            plsc.store_compressed(cd_v.at[pl.ds(cnt, 16)], packed, mask=msk)
            plsc.store_compressed(ch_v.at[pl.ds(cnt, 16)],
                                  lax.shift_right_logical(s16, 1), mask=msk)
            return cnt + plsc.all_reduce_population_count(msk)[0]
        m = lax.fori_loop(0, CC // 16, _cmp, jnp.int32(0), unroll=8)

        def _pad(p, _):
            cd_v[pl.ds(m + p * 16, 16)] = izeros
            ch_v[pl.ds(m + p * 16, 16)] = izeros
            return 0
        lax.fori_loop(0, SUB // 16, _pad, 0)
        return m

    def issue_gather(i0):
        for j in range(GK):
            pltpu.async_copy(h_sh.at[ch_v.at[pl.ds(i0 + j * GR, GR)]],
                             rows_v.at[pl.ds(j * GR, GR)], gsem)

    def wait_gather(i0):
        for j in range(GK):
            pltpu.make_async_copy(h_sh.at[ch_v.at[pl.ds(i0 + j * GR, GR)]],
                                  rows_v.at[pl.ds(j * GR, GR)], gsem).wait()

    def wcompute(i0, m):
        def _wg(g, _):
            csl = pl.ds(i0 + g * 16, 16)
            raw16 = cd_v[csl]
            dl16 = raw16 & 4095
            s16 = (ch_v[csl] << 1) | (lax.shift_right_logical(raw16, 12) & 1)
            x = (plsc.load_gather(s1r_v, [dl16])
                 + plsc.load_gather(s2_v, [s16]))
            x = jnp.where(x >= 0.0, x, 0.01 * x)
            x = jnp.minimum(jnp.maximum(x, -2.0), 2.0)
            lane = i0 + g * 16 + lax.iota(jnp.int32, 16)
            w_v[pl.ds(g * 16, 16)] = jnp.where(lane < m, jnp.exp(x), 0.0)
            return 0
        lax.fori_loop(0, SUB // 16, _wg, 0, unroll=4)

    def accum(i0, n):
        def _edge(e, raw):
            raw_next = cd_v[pl.ds(i0 + e + 1, 16)][0]
            dloc = raw & 4095
            coff = lax.shift_right_logical(raw, 12) * 64
            wv = jnp.full((16,), w_v[pl.ds(e, 16)][0], jnp.float32)
            prods = []
            for j in range(D // 32):
                x32 = plsc.bitcast(rows_v[e, pl.ds(coff + j * 16, 16)],
                                   jnp.bfloat16)
                a, b = plsc.unpack(x32, format=plsc.PackFormat.INTERLEAVED)
                prods += [a * wv, b * wv]
            plsc.addupdate(den_v.at[pl.ds(dloc, 16)],
                           jnp.where(lane0, wv, 0.0))
            for j in range(D // 16):
                plsc.addupdate(acc_v.at[dloc, pl.ds(j * 16, 16)], prods[j])
            return raw_next
        lax.fori_loop(0, n, _edge, cd_v[pl.ds(i0, 16)][0])

    issue_edges(0)

    def _chunk(c, _):
        wait_edges()
        m = compact()

        @pl.when(c + 1 < N_CHUNKS)
        def _():
            issue_edges(c + 1)
        issue_gather(0)
        wcompute(0, m)
        wait_gather(0)
        accum(0, jnp.minimum(m, SUB))

        nsub = (m + SUB - 1) // SUB

        def _slow(q, _):
            i0 = q * SUB
            issue_gather(i0)
            wcompute(i0, m)
            wait_gather(i0)
            accum(i0, jnp.minimum(m - i0, SUB))
            return 0
        lax.fori_loop(1, nsub, _slow, 0)
        return 0

    lax.fori_loop(0, N_CHUNKS, _chunk, 0)

    pltpu.sync_copy(acc_v, num_out.at[pl.ds(lo, RPT)])
    pltpu.sync_copy(den_v.at[pl.ds(0, RPT)], den_out.at[pl.ds(lo, RPT)])


def _sc_accum(h, s1p, s2p, dst, src):
    mesh = plsc.VectorSubcoreMesh(core_axis_name="c", subcore_axis_name="s",
                                  num_cores=NC)
    fn = functools.partial(
        pl.kernel,
        mesh=mesh,
        out_type=(
            jax.ShapeDtypeStruct((N_PAD, D), jnp.float32),
            jax.ShapeDtypeStruct((N_PAD,), jnp.float32),
        ),
        scratch_types=[
            pltpu.VMEM((RPT,), jnp.float32),
            pltpu.VMEM((N_PAD,), jnp.float32),
            pltpu.VMEM((CC,), jnp.int32),
            pltpu.VMEM((CC,), jnp.int32),
            pltpu.VMEM((CCP,), jnp.int32),
            pltpu.VMEM((CCP,), jnp.int32),
            pltpu.VMEM((SUB + 16,), jnp.float32),
            pltpu.VMEM((SUB, D), jnp.int32),
            pltpu.VMEM((RPT, D), jnp.float32),
            pltpu.VMEM((RPT + 16,), jnp.float32),
            pltpu.VMEM_SHARED((NPAIR, D), jnp.int32),
            pltpu.SemaphoreType.DMA,
            pltpu.SemaphoreType.DMA,
        ],
        compiler_params=pltpu.CompilerParams(needs_layout_passes=False),
    )(_sc_body)
    return fn(h, s1p, s2p, dst, src)


def _fin_body(num_ref, den_ref, out_ref):
    den = den_ref[...]
    den = jnp.where(den > 0.0, den, 1.0)
    out_ref[...] = num_ref[...] / den


def _finish(num, den):
    blk = 1000
    grid = N_NODES // blk
    return pl.pallas_call(
        _fin_body,
        grid=(grid,),
        in_specs=[
            pl.BlockSpec((blk, D), lambda i: (i, 0)),
            pl.BlockSpec((blk, 1), lambda i: (i, 0)),
        ],
        out_specs=pl.BlockSpec((blk, D), lambda i: (i, 0)),
        out_shape=jax.ShapeDtypeStruct((N_NODES, D), jnp.float32),
    )(num, den)


def kernel(node_states, edges, kernel, kernel_attention):
    a = kernel_attention[:, 0]
    a8 = jnp.stack([a[:D], a[D:]] + [jnp.zeros((D,), jnp.float32)] * 6,
                   axis=1)
    g = np.empty((D,), np.int32)
    for j in range(D // 32):
        for t in range(16):
            g[32 * j + 2 * t] = 32 * j + t
            g[32 * j + 2 * t + 1] = 32 * j + 16 + t
    wp = kernel[:, g]
    h, s1, s2 = _prep(node_states, kernel, wp, a8)
    h = lax.bitcast_convert_type(h.reshape(N_NODES, D // 2, 2), jnp.int32)
    h = h.reshape(NPAIR, D)
    s1p = jnp.pad(s1, (0, N_PAD - N_NODES))
    s2p = jnp.pad(s2, (0, N_PAD - N_NODES))
    dst = edges[:, 0].astype(jnp.int32)
    src = edges[:, 1].astype(jnp.int32)
    num, den = _sc_accum(h, s1p, s2p, dst, src)
    return _finish(num, den.reshape(N_PAD, 1))
